# baseline (speedup 1.0000x reference)
import functools

import jax
import jax.numpy as jnp
from jax import lax
from jax.experimental import pallas as pl
from jax.experimental.pallas import tpu as pltpu
from jax.experimental.pallas import tpu_sc as plsc

_NNZ = 320000
_NSEG = 10000
_NSEGP = 10240
_D = 128
_L = 16
_NC = 2
_NS = 16
_NW = _NC * _NS

_RNG = 320
_NPASS = 2
_PSEG = _NSEGP // _NPASS
_S = 4000
_NSC = _NNZ // _S
_QPC = _S // _L
_UNR = 5
_FB = 128
_ECAP = 4160

_CHUNK = 80
_E2 = _NNZ // _NW
_NCH2 = _E2 // _CHUNK

_mesh = plsc.VectorSubcoreMesh(core_axis_name="c", subcore_axis_name="s")


def _prefix_sum(v, lanes):
    cs = v
    for k in (1, 2, 4, 8):
        g = jnp.take(cs, (lanes - k) & (_L - 1), axis=0)
        cs = cs + jnp.where(lanes >= k, g, 0)
    return cs


def _lower_bound(cs, lanes):
    tgt = lanes + 1
    pos = jnp.zeros((_L,), jnp.int32)
    for s in (8, 4, 2, 1):
        cand = pos + s
        cv = jnp.take(cs, cand - 1, axis=0)
        pos = jnp.where(cv < tgt, cand, pos)
    return pos


@functools.partial(
    pl.kernel,
    out_type=(
        jax.ShapeDtypeStruct((_NSEGP, _D), jnp.float32),
        jax.ShapeDtypeStruct((_NSEGP, _D), jnp.float32),
    ),
    mesh=_mesh,
    scratch_types=[
        pltpu.VMEM((_RNG, _D), jnp.float32),
        pltpu.VMEM((_RNG, _L), jnp.float32),
        pltpu.VMEM((_FB, _D), jnp.float32),
        pltpu.VMEM((_FB,), jnp.int32),
        pltpu.VMEM((_S,), jnp.int32),
        pltpu.VMEM((_S,), jnp.int32),
        pltpu.VMEM((_ECAP,), jnp.int32),
        pltpu.VMEM((_ECAP,), jnp.int32),
        pltpu.VMEM((_L,), jnp.float32),
        pltpu.SemaphoreType.DMA,
        pltpu.SemaphoreType.DMA,
        pltpu.SemaphoreType.DMA,
    ],
)
def _segment_means(rows, cols, values, zd, mrow_out, mcol_out,
                   acc, accc, gbuf, gidx, ibufa, ibufb, elist, rlist, zd_v,
                   sem, sema, semb):
    core = lax.axis_index("c")
    tid = lax.axis_index("s")

    zf = jnp.zeros((_L,), jnp.float32)
    zi = jnp.zeros((_L,), jnp.int32)
    onesf = jnp.ones((_L,), jnp.float32)
    lanes = lax.iota(jnp.int32, _L)

    def zero_elist(i, _):
        elist[pl.ds(i * _L, _L)] = zi
        return 0

    lax.fori_loop(0, _ECAP // _L, zero_elist, 0)

    pltpu.sync_copy(zd, zd_v)
    zc = zd_v[...]

    def accum_row(row, gr):
        for kk in range(_D // _L):
            sl = pl.ds(kk * _L, _L)
            acc[row, sl] = acc[row, sl] + gbuf[gr, sl]
        accc[row, :] = accc[row, :] + onesf

    def sweep(idx_hbm, out_hbm):
      def pass_body(pp, _):
        base = pp * _PSEG + tid * _RNG

        def zero_acc(i, _):
            acc[i // 8, pl.ds((i % 8) * _L, _L)] = zf
            return 0

        lax.fori_loop(0, _RNG * 8, zero_acc, 0)

        def zero_accc(i, _):
            accc[i, :] = zf
            return 0

        lax.fori_loop(0, _RNG, zero_accc, 0)

        def flush_batch(fo):
            for q in range(_FB // _L):
                gidx[pl.ds(q * _L, _L)] = elist[pl.ds(fo + q * _L, _L)]
            pltpu.async_copy(values.at[gidx], gbuf, sem).wait()

        def scan_chunk(c, buf, off):
            def scanq(u, off):
                for j in range(_UNR):
                    q = u * _UNR + j
                    iv = buf[pl.ds(q * _L, _L)]
                    rel = iv - base
                    m = (rel >= 0) & (rel < _RNG)
                    mi = jnp.where(m, 1, 0).astype(jnp.int32)
                    cs = _prefix_sum(mi, lanes)
                    cnt = cs[_L - 1]
                    sel = _lower_bound(cs, lanes)
                    ec = jnp.take(c * _S + q * _L + lanes, sel, axis=0)
                    rc = jnp.take(rel, sel, axis=0)
                    elist[pl.ds(off, _L)] = ec
                    rlist[pl.ds(off, _L)] = rc
                    off = off + cnt
                return off

            off = lax.fori_loop(0, _QPC // _UNR, scanq, off)

            nfull = off // _FB

            def flush(i, _):
                fo = i * _FB
                flush_batch(fo)

                def grp(g, _):
                    rels = rlist[pl.ds(fo + g * _L, _L)]
                    for l in range(_L):
                        accum_row(rels[l], g * _L + l)
                    return 0

                lax.fori_loop(0, _FB // _L, grp, 0)
                return 0

            lax.fori_loop(0, nfull, flush, 0)

            rem_base = nfull * _FB
            for q in range(_FB // _L):
                elist[pl.ds(q * _L, _L)] = elist[pl.ds(rem_base + q * _L, _L)]
                rlist[pl.ds(q * _L, _L)] = rlist[pl.ds(rem_base + q * _L, _L)]
            return off - rem_base

        def islice(c):
            return idx_hbm.at[pl.ds(c * _S, _S)]

        pltpu.async_copy(islice(0), ibufa, sema)

        def body2(i, off):
            c0 = 2 * i
            pltpu.async_copy(islice(c0 + 1), ibufb, semb)
            pltpu.make_async_copy(islice(c0), ibufa, sema).wait()
            off = scan_chunk(c0, ibufa, off)

            @pl.when(i < _NSC // 2 - 1)
            def _():
                pltpu.async_copy(islice(c0 + 2), ibufa, sema)

            pltpu.make_async_copy(islice(c0 + 1), ibufb, semb).wait()
            off = scan_chunk(c0 + 1, ibufb, off)
            return off

        nrem = lax.fori_loop(0, _NSC // 2, body2, 0)

        flush_batch(0)

        def tgrp(g, _):
            rels = rlist[pl.ds(g * _L, _L)]
            for l in range(_L):
                @pl.when(g * _L + l < nrem)
                def _():
                    accum_row(rels[l], g * _L + l)
            return 0

        lax.fori_loop(0, _FB // _L, tgrp, 0)

        def meanrow(s, _):
            r = 1.0 / jnp.maximum(accc[s, pl.ds(0, _L)], 1.0)
            for kk in range(_D // _L):
                sl = pl.ds(kk * _L, _L)
                acc[s, sl] = acc[s, sl] * r - zc
            return 0

        lax.fori_loop(0, _RNG, meanrow, 0)
        pltpu.sync_copy(acc, out_hbm.at[pl.ds(base, _RNG), :])
        return 0

      lax.fori_loop(0, _NPASS, pass_body, 0)

    @pl.when(core == 0)
    def _():
        sweep(rows, mrow_out)

    @pl.when(core == 1)
    def _():
        sweep(cols, mcol_out)


@functools.partial(
    pl.kernel,
    out_type=jax.ShapeDtypeStruct((_NNZ, 2 * _D), jnp.float32),
    mesh=_mesh,
    scratch_types=[
        pltpu.VMEM((_CHUNK,), jnp.int32),
        pltpu.VMEM((_CHUNK,), jnp.int32),
        pltpu.VMEM((_CHUNK, _D), jnp.float32),
        pltpu.VMEM((_CHUNK, _D), jnp.float32),
        pltpu.VMEM((_CHUNK, _D), jnp.float32),
        pltpu.VMEM((_CHUNK, 2 * _D), jnp.float32),
        pltpu.SemaphoreType.DMA,
        pltpu.SemaphoreType.DMA,
    ],
)
def _center(rows, cols, values, mrow, mcol, out,
            idxr, idxc, g0, g1, vals_v, obuf, sem0, sem1):
    core = lax.axis_index("c")
    tid = lax.axis_index("s")
    wid = tid * _NC + core
    base = wid * _E2

    def body(ch, _):
        start = base + ch * _CHUNK
        pltpu.sync_copy(rows.at[pl.ds(start, _CHUNK)], idxr)
        pltpu.sync_copy(cols.at[pl.ds(start, _CHUNK)], idxc)
        cp1 = pltpu.async_copy(mrow.at[idxr], g1, sem0)
        cp0 = pltpu.async_copy(mcol.at[idxc], g0, sem1)
        pltpu.sync_copy(values.at[pl.ds(start, _CHUNK), :], vals_v)
        cp1.wait()
        cp0.wait()

        def row_body(r, _):
            for k in range(_D // _L):
                sl = pl.ds(k * _L, _L)
                v = vals_v[r, sl]
                obuf[r, sl] = v - g0[r, sl]
                obuf[r, pl.ds(_D + k * _L, _L)] = v - g1[r, sl]
            return 0

        lax.fori_loop(0, _CHUNK, row_body, 0)
        pltpu.sync_copy(obuf, out.at[pl.ds(start, _CHUNK), :])
        return 0

    lax.fori_loop(0, _NCH2, body, 0)


def kernel(values, indices, n0, n1):
    rows = indices[0]
    cols = indices[1]
    zero_dep = (jnp.asarray(n0 - _NSEG + n1 - _NSEG)).astype(values.dtype)
    zd = jnp.broadcast_to(zero_dep, (_L,))
    mrow, mcol = _segment_means(rows, cols, values, zd)
    return _center(rows, cols, values, mrow, mcol)


# speedup vs baseline: 1.5089x; 1.5089x over previous
import functools

import jax
import jax.numpy as jnp
from jax import lax
from jax.experimental import pallas as pl
from jax.experimental.pallas import tpu as pltpu
from jax.experimental.pallas import tpu_sc as plsc

_NNZ = 320000
_NSEG = 10000
_NSEGP = 10240
_D = 128
_L = 16
_NC = 2
_NS = 16
_NW = _NC * _NS

_RNG = 320
_NPASS = 2
_PSEG = _NSEGP // _NPASS
_S = 4000
_NSC = _NNZ // _S
_QPC = _S // _L
_UNR = 5
_FB = 128
_ECAP = 4160

_CHUNK = 80
_E2 = _NNZ // _NW
_NCH2 = _E2 // _CHUNK

_mesh = plsc.VectorSubcoreMesh(core_axis_name="c", subcore_axis_name="s")


def _prefix_sum(v, lanes):
    cs = v
    for k in (1, 2, 4, 8):
        g = jnp.take(cs, (lanes - k) & (_L - 1), axis=0)
        cs = cs + jnp.where(lanes >= k, g, 0)
    return cs


def _lower_bound(cs, lanes):
    tgt = lanes + 1
    pos = jnp.zeros((_L,), jnp.int32)
    for s in (8, 4, 2, 1):
        cand = pos + s
        cv = jnp.take(cs, cand - 1, axis=0)
        pos = jnp.where(cv < tgt, cand, pos)
    return pos


@functools.partial(
    pl.kernel,
    out_type=(
        jax.ShapeDtypeStruct((_NSEGP, _D), jnp.float32),
        jax.ShapeDtypeStruct((_NSEGP, _D), jnp.float32),
    ),
    mesh=_mesh,
    scratch_types=[
        pltpu.VMEM((_RNG, _D), jnp.float32),
        pltpu.VMEM((_RNG, _L), jnp.float32),
        pltpu.VMEM((_FB, _D), jnp.float32),
        pltpu.VMEM((_FB,), jnp.int32),
        pltpu.VMEM((_S,), jnp.int32),
        pltpu.VMEM((_S,), jnp.int32),
        pltpu.VMEM((_ECAP,), jnp.int32),
        pltpu.VMEM((_ECAP,), jnp.int32),
        pltpu.VMEM((_S,), jnp.int32),
        pltpu.VMEM((_S,), jnp.int32),
        pltpu.VMEM((_S,), jnp.int32),
        pltpu.VMEM((_L,), jnp.float32),
        pltpu.SemaphoreType.DMA,
        pltpu.SemaphoreType.DMA,
        pltpu.SemaphoreType.DMA,
    ],
)
def _segment_means(rows, cols, values, zd, mrow_out, mcol_out,
                   acc, accc, gbuf, gidx, ibufa, ibufb, elist, rlist,
                   eraw, rraw, csraw, zd_v, sem, sema, semb):
    core = lax.axis_index("c")
    tid = lax.axis_index("s")

    zf = jnp.zeros((_L,), jnp.float32)
    zi = jnp.zeros((_L,), jnp.int32)
    onesf = jnp.ones((_L,), jnp.float32)
    lanes = lax.iota(jnp.int32, _L)

    def zero_elist(i, _):
        elist[pl.ds(i * _L, _L)] = zi
        return 0

    lax.fori_loop(0, _ECAP // _L, zero_elist, 0)

    pltpu.sync_copy(zd, zd_v)
    zc = zd_v[...]

    def accum_row(row, gr):
        for kk in range(_D // _L):
            sl = pl.ds(kk * _L, _L)
            acc[row, sl] = acc[row, sl] + gbuf[gr, sl]
        accc[row, :] = accc[row, :] + onesf

    def sweep(idx_hbm, out_hbm):
      def pass_body(pp, _):
        base = pp * _PSEG + tid * _RNG

        def zero_acc(i, _):
            acc[i // 8, pl.ds((i % 8) * _L, _L)] = zf
            return 0

        lax.fori_loop(0, _RNG * 8, zero_acc, 0)

        def zero_accc(i, _):
            accc[i, :] = zf
            return 0

        lax.fori_loop(0, _RNG, zero_accc, 0)

        def flush_batch(fo):
            for q in range(_FB // _L):
                gidx[pl.ds(q * _L, _L)] = elist[pl.ds(fo + q * _L, _L)]
            pltpu.async_copy(values.at[gidx], gbuf, sem).wait()

        def scan_chunk(c, buf, off):
            def stage1(q):
                iv = buf[pl.ds(q * _L, _L)]
                rel = iv - base
                m = (rel >= 0) & (rel < _RNG)
                mi = jnp.where(m, 1, 0).astype(jnp.int32)
                cs = _prefix_sum(mi, lanes)
                sel = _lower_bound(cs, lanes)
                ec = jnp.take(c * _S + q * _L + lanes, sel, axis=0)
                rc = jnp.take(rel, sel, axis=0)
                eraw[pl.ds(q * _L, _L)] = ec
                rraw[pl.ds(q * _L, _L)] = rc
                csraw[pl.ds(q * _L, _L)] = cs

            plsc.parallel_loop(0, _QPC, 1, unroll=8)(stage1)

            def merge(q, off):
                cnt = csraw[pl.ds(q * _L, _L)][_L - 1]
                elist[pl.ds(off, _L)] = eraw[pl.ds(q * _L, _L)]
                rlist[pl.ds(off, _L)] = rraw[pl.ds(q * _L, _L)]
                return off + cnt

            off = lax.fori_loop(0, _QPC, merge, off)

            nfull = off // _FB

            def flush(i, _):
                fo = i * _FB
                flush_batch(fo)

                def grp(g, _):
                    rels = rlist[pl.ds(fo + g * _L, _L)]
                    for l in range(_L):
                        accum_row(rels[l], g * _L + l)
                    return 0

                lax.fori_loop(0, _FB // _L, grp, 0)
                return 0

            lax.fori_loop(0, nfull, flush, 0)

            rem_base = nfull * _FB
            for q in range(_FB // _L):
                elist[pl.ds(q * _L, _L)] = elist[pl.ds(rem_base + q * _L, _L)]
                rlist[pl.ds(q * _L, _L)] = rlist[pl.ds(rem_base + q * _L, _L)]
            return off - rem_base

        def islice(c):
            return idx_hbm.at[pl.ds(c * _S, _S)]

        pltpu.async_copy(islice(0), ibufa, sema)

        def body2(i, off):
            c0 = 2 * i
            pltpu.async_copy(islice(c0 + 1), ibufb, semb)
            pltpu.make_async_copy(islice(c0), ibufa, sema).wait()
            off = scan_chunk(c0, ibufa, off)

            @pl.when(i < _NSC // 2 - 1)
            def _():
                pltpu.async_copy(islice(c0 + 2), ibufa, sema)

            pltpu.make_async_copy(islice(c0 + 1), ibufb, semb).wait()
            off = scan_chunk(c0 + 1, ibufb, off)
            return off

        nrem = lax.fori_loop(0, _NSC // 2, body2, 0)

        flush_batch(0)

        def tgrp(g, _):
            rels = rlist[pl.ds(g * _L, _L)]
            for l in range(_L):
                @pl.when(g * _L + l < nrem)
                def _():
                    accum_row(rels[l], g * _L + l)
            return 0

        lax.fori_loop(0, _FB // _L, tgrp, 0)

        def meanrow(s):
            r = 1.0 / jnp.maximum(accc[s, pl.ds(0, _L)], 1.0)
            for kk in range(_D // _L):
                sl = pl.ds(kk * _L, _L)
                acc[s, sl] = acc[s, sl] * r - zc

        plsc.parallel_loop(0, _RNG, 1, unroll=4)(meanrow)
        pltpu.sync_copy(acc, out_hbm.at[pl.ds(base, _RNG), :])
        return 0

      lax.fori_loop(0, _NPASS, pass_body, 0)

    @pl.when(core == 0)
    def _():
        sweep(rows, mrow_out)

    @pl.when(core == 1)
    def _():
        sweep(cols, mcol_out)


@functools.partial(
    pl.kernel,
    out_type=jax.ShapeDtypeStruct((_NNZ, 2 * _D), jnp.float32),
    mesh=_mesh,
    scratch_types=[
        pltpu.VMEM((_CHUNK,), jnp.int32),
        pltpu.VMEM((_CHUNK,), jnp.int32),
        pltpu.VMEM((_CHUNK, _D), jnp.float32),
        pltpu.VMEM((_CHUNK, _D), jnp.float32),
        pltpu.VMEM((_CHUNK, _D), jnp.float32),
        pltpu.VMEM((_CHUNK, 2 * _D), jnp.float32),
        pltpu.VMEM((_CHUNK, 2 * _D), jnp.float32),
        pltpu.SemaphoreType.DMA,
        pltpu.SemaphoreType.DMA,
        pltpu.SemaphoreType.DMA,
        pltpu.SemaphoreType.DMA,
    ],
)
def _center(rows, cols, values, mrow, mcol, out,
            idxr, idxc, g0, g1, vals_v, obufa, obufb, sem0, sem1,
            osema, osemb):
    core = lax.axis_index("c")
    tid = lax.axis_index("s")
    wid = tid * _NC + core
    base = wid * _E2

    def do_chunk(ch, ob, osem, drain):
        start = base + ch * _CHUNK
        cpr = pltpu.async_copy(rows.at[pl.ds(start, _CHUNK)], idxr, sem0)
        cpc = pltpu.async_copy(cols.at[pl.ds(start, _CHUNK)], idxc, sem1)
        cpr.wait()
        cpc.wait()
        cp1 = pltpu.async_copy(mrow.at[idxr], g1, sem0)
        cp0 = pltpu.async_copy(mcol.at[idxc], g0, sem1)
        pltpu.sync_copy(values.at[pl.ds(start, _CHUNK), :], vals_v)
        drain()
        cp1.wait()
        cp0.wait()

        def row_body(r):
            for k in range(_D // _L):
                sl = pl.ds(k * _L, _L)
                v = vals_v[r, sl]
                ob[r, sl] = v - g0[r, sl]
                ob[r, pl.ds(_D + k * _L, _L)] = v - g1[r, sl]

        plsc.parallel_loop(0, _CHUNK, 1, unroll=4)(row_body)
        pltpu.async_copy(ob, out.at[pl.ds(start, _CHUNK), :], osem)

    def drain_ob(ob, osem):
        pltpu.make_async_copy(ob, out.at[pl.ds(base, _CHUNK), :], osem).wait()

    do_chunk(0, obufa, osema, lambda: None)

    def body(i, _):
        def drain_a():
            drain_ob(obufa, osema)

        def drain_b():
            @pl.when(i > 0)
            def _():
                drain_ob(obufb, osemb)

        do_chunk(1 + 2 * i, obufb, osemb, drain_b)
        do_chunk(2 + 2 * i, obufa, osema, drain_a)
        return 0

    lax.fori_loop(0, (_NCH2 - 1) // 2, body, 0)
    drain_ob(obufa, osema)
    drain_ob(obufb, osemb)


def kernel(values, indices, n0, n1):
    rows = indices[0]
    cols = indices[1]
    zero_dep = (jnp.asarray(n0 - _NSEG + n1 - _NSEG)).astype(values.dtype)
    zd = jnp.broadcast_to(zero_dep, (_L,))
    mrow, mcol = _segment_means(rows, cols, values, zd)
    return _center(rows, cols, values, mrow, mcol)
